# Initial kernel scaffold; baseline (speedup 1.0000x reference)
import jax
import jax.numpy as jnp
from jax.experimental import pallas as pl


def kernel(xyz, new_xyz, features):
    raise NotImplementedError("write your pallas kernel here")


# speedup vs baseline: 28.0783x; 28.0783x over previous
import dataclasses
import functools

import numpy as np
import jax
import jax.numpy as jnp
from jax import lax
from jax.experimental import pallas as pl
from jax.experimental.pallas import tpu as pltpu
from jax.experimental.pallas import tpu_sc as plsc

_RADIUS2 = np.float32(0.2 * 0.2)
_K = 32
_L = 16
_NC = 2
_NS = 16


def _qag(xyzt, nxt, features):
    B, _, N = xyzt.shape
    S = nxt.shape[2]
    C = features.shape[1]
    K = _K
    NW = _NC * _NS
    QW = (B * S) // NW
    WPB = NW // B
    CW = C // WPB
    HALF = (S * K) // 2

    mesh = plsc.VectorSubcoreMesh(core_axis_name="c", subcore_axis_name="s")
    cp = pltpu.CompilerParams()
    if "needs_layout_passes" in pltpu.CompilerParams.__dataclass_fields__:
        cp = dataclasses.replace(cp, needs_layout_passes=False)

    @functools.partial(
        pl.kernel,
        out_type=jax.ShapeDtypeStruct((B, 3 + C, S * K), jnp.float32),
        mesh=mesh,
        compiler_params=cp,
        scratch_types=[
            pltpu.VMEM((3, N), jnp.float32),
            pltpu.VMEM((3, QW), jnp.float32),
            pltpu.VMEM((3 * _L,), jnp.int32),
            pltpu.VMEM((N,), jnp.float32),
            pltpu.VMEM((2 * QW * K,), jnp.float32),
            pltpu.VMEM((S * K,), jnp.int32),
            pltpu.VMEM_SHARED((2, S * K), jnp.int32),
        ],
    )
    def qag(xyzt_hbm, nxt_hbm, feat_hbm, out_hbm,
            pts, q, idxbuf, row, ostage, idx_all, shidx):
        iota16 = lax.iota(jnp.int32, _L)

        def _splat(v, lane):
            sel = jnp.where(iota16 == lane, v, jnp.zeros_like(v))
            return jnp.full((_L,), jnp.sum(sel), dtype=v.dtype)

        wid = lax.axis_index("c") * _NS + lax.axis_index("s")
        b = wid // WPB
        slot = b % 2
        qoff = (wid % WPB) * QW

        pltpu.sync_copy(xyzt_hbm.at[b], pts)
        pltpu.sync_copy(nxt_hbm.at[b, :, pl.ds(qoff, QW)], q)

        @pl.loop(0, QW)
        def _per_query(qi):
            g = (qi // _L) * _L
            lane = qi - g
            qx = _splat(q[0, pl.ds(g, _L)], lane)
            qy = _splat(q[1, pl.ds(g, _L)], lane)
            qz = _splat(q[2, pl.ds(g, _L)], lane)
            idxbuf[pl.ds(0, _L)] = jnp.zeros((_L,), jnp.int32)

            def cond(carry):
                off, cnt = carry
                return jnp.logical_and(cnt < K, off < N)

            def step(carry):
                off, cnt = carry
                xv = pts[0, pl.ds(off, _L)]
                yv = pts[1, pl.ds(off, _L)]
                zv = pts[2, pl.ds(off, _L)]
                dx = qx - xv
                dy = qy - yv
                dz = qz - zv
                d2 = dx * dx + dy * dy + dz * dz
                m = d2 < _RADIUS2
                plsc.store_compressed(idxbuf.at[pl.ds(cnt, _L)],
                                      iota16 + off, mask=m)
                hits = jnp.sum(jnp.where(m, 1, 0))
                return off + _L, cnt + hits

            _, cnt = lax.while_loop(cond, step, (jnp.int32(0), jnp.int32(0)))

            k0 = idxbuf[pl.ds(0, _L)]
            k1 = idxbuf[pl.ds(_L, _L)]
            first = _splat(k0, jnp.int32(0))
            cntv = jnp.full((_L,), cnt, jnp.int32)
            f0 = jnp.where(iota16 < cntv, k0, first)
            f1 = jnp.where(iota16 + _L < cntv, k1, first)
            idx_all[pl.ds((qoff + qi) * K, _L)] = f0
            idx_all[pl.ds((qoff + qi) * K + _L, _L)] = f1
            for d in range(3):
                dv = jnp.full((_L,), d, jnp.int32)
                g0 = plsc.load_gather(pts, [dv, f0])
                g1 = plsc.load_gather(pts, [dv, f1])
                qd = (qx, qy, qz)[d]
                st = row if d == 2 else ostage
                base = 0 if d == 2 else d * QW * K
                st[pl.ds(base + qi * K, _L)] = g0 - qd
                st[pl.ds(base + qi * K + _L, _L)] = g1 - qd

        for d in range(3):
            st = row if d == 2 else ostage
            base = 0 if d == 2 else d * QW * K
            pltpu.sync_copy(st.at[pl.ds(base, QW * K)],
                            out_hbm.at[b, d, pl.ds(qoff * K, QW * K)])
        pltpu.sync_copy(idx_all.at[pl.ds(qoff * K, QW * K)],
                        shidx.at[slot, pl.ds(qoff * K, QW * K)])
        plsc.subcore_barrier()

        pltpu.sync_copy(shidx.at[slot], idx_all)
        coff = (wid % WPB) * CW

        QTR = (S * K) // 4

        @pl.loop(0, CW)
        def _per_channel(ci):
            c = coff + ci
            pltpu.sync_copy(feat_hbm.at[b, c], row)
            for quarter in range(4):
                @pl.loop(0, QTR, step=8 * _L)
                def _gather(i):
                    for u in range(8):
                        o = i + u * _L
                        iv = idx_all[pl.ds(quarter * QTR + o, _L)]
                        ostage[pl.ds(o, _L)] = plsc.load_gather(row, [iv])

                pltpu.sync_copy(ostage.at[pl.ds(0, QTR)],
                                out_hbm.at[b, 3 + c, pl.ds(quarter * QTR, QTR)])

    return qag(xyzt, nxt, features)


def kernel(xyz, new_xyz, features):
    B, N, _ = xyz.shape
    S = new_xyz.shape[1]
    C = features.shape[1]
    xyzt = jnp.transpose(xyz, (0, 2, 1))
    nxt = jnp.transpose(new_xyz, (0, 2, 1))
    out = _qag(xyzt, nxt, features)
    return out.reshape(B, 3 + C, S, _K)
